# baseline (speedup 1.0000x reference)
import functools

import jax
import jax.numpy as jnp
from jax import lax
from jax.experimental import pallas as pl
from jax.experimental.pallas import tpu as pltpu
from jax.experimental.pallas import tpu_sc as plsc

_N = 16777216
_NB = 17
_NC = 2
_NS = 16
_NW = _NC * _NS
_PER_W = _N // _NW
_CH = 16384
_NCH = _PER_W // _CH

_mesh = plsc.VectorSubcoreMesh(core_axis_name="c", subcore_axis_name="s")


@functools.partial(
    pl.kernel,
    out_type=jax.ShapeDtypeStruct((_NW * 16,), jnp.int32),
    mesh=_mesh,
    scratch_types=[
        pltpu.VMEM((_CH,), jnp.float32),
        pltpu.VMEM((_NB * 16,), jnp.float32),
        pltpu.VMEM((16,), jnp.int32),
    ],
    compiler_params=pltpu.CompilerParams(needs_layout_passes=False),
)
def _sc_bucketize_check(vals_hbm, b_hbm, out_hbm, buf, bvm, okv):
    cid = lax.axis_index("c")
    sid = lax.axis_index("s")
    wid = sid * _NC + cid
    base = wid * _PER_W

    pltpu.sync_copy(b_hbm, bvm)
    bvecs = [bvm[pl.ds(j * 16, 16)] for j in range(_NB)]

    def chunk_body(c, ok):
        pltpu.sync_copy(vals_hbm.at[pl.ds(base + c * _CH, _CH)], buf)

        def vreg_body(i, ok):
            v = buf[pl.ds(i * 16, 16)]
            idx1 = jnp.zeros((16,), jnp.int32)
            idx2 = jnp.zeros((16,), jnp.int32)
            for j in range(_NB):
                idx1 = idx1 + (bvecs[j] < v).astype(jnp.int32)
            for j in reversed(range(_NB)):
                idx2 = idx2 + (v <= bvecs[j]).astype(jnp.int32)
            idx2 = _NB - idx2
            return ok & (idx1 == idx2).astype(jnp.int32)

        return lax.fori_loop(0, _CH // 16, vreg_body, ok)

    ok = lax.fori_loop(0, _NCH, chunk_body, jnp.ones((16,), jnp.int32))
    okv[...] = ok
    pltpu.sync_copy(okv, out_hbm.at[pl.ds(wid * 16, 16)])


def kernel(vals, boundaries):
    b_rep = jnp.repeat(boundaries, 16)
    flags = _sc_bucketize_check(vals, b_rep)
    return jnp.all(flags == 1)
